# baseline (speedup 1.0000x reference)
import functools

import jax
import jax.numpy as jnp
import numpy as np
from jax import lax
from jax.experimental import pallas as pl
from jax.experimental.pallas import tpu as pltpu
from jax.experimental.pallas import tpu_sc as plsc

EMB = 128
N_NODES = 10000
N_EDGES = 320000

NC = 2
NS = 16
NW = NC * NS
K = 128
NCH = 80
EPW = NCH * K
E_PAD = NW * EPW
N_ACC = 10112
ROWS_PER_TILE = N_ACC // NS
QROWS = 32
NEG = -1.0e30

def _tnode_body(x_ref, w_ref, o_ref):
    o_ref[...] = jnp.dot(x_ref[...], w_ref[...],
                         preferred_element_type=jnp.float32)


EB = 8192


def _efeat_body(ea_ref, et_ref, wa_ref, wt_ref, b_ref, o_ref):
    i = pl.program_id(0)
    y = (jnp.dot(ea_ref[...], wa_ref[...], preferred_element_type=jnp.float32)
         + jnp.dot(et_ref[...], wt_ref[...], preferred_element_type=jnp.float32)
         + b_ref[...])
    rows = jax.lax.broadcasted_iota(jnp.int32, (EB, 1), 0) + i * EB
    o_ref[...] = jnp.where(rows >= N_EDGES, NEG, y)


def _sc_body(tnode_hbm, efeat_hbm, pidx_hbm, part_hbm,
             pidx_v0, pidx_v1, src_v0, src_v1, dst_q, t_v0, t_v1, e_v,
             acc_sh, sem_t0, sem_t1, sem_e, sem_p0, sem_p1):
    c = lax.axis_index("c")
    s = lax.axis_index("s")
    wid = c * NS + s
    t_v = (t_v0, t_v1)
    src_v = (src_v0, src_v1)
    pidx_v = (pidx_v0, pidx_v1)
    sem_t = (sem_t0, sem_t1)
    sem_p = (sem_p0, sem_p1)
    base = wid * EPW

    zvec = jnp.zeros((16,), jnp.float32)

    @pl.loop(0, K)
    def _zero_rows(r):
        for gg in range(EMB // 16):
            e_v[r, pl.ds(gg * 16, 16)] = zvec

    row0 = s * ROWS_PER_TILE
    nfull = ROWS_PER_TILE // K
    rem = ROWS_PER_TILE - nfull * K
    for bq in range(nfull):
        pltpu.sync_copy(e_v, acc_sh.at[pl.ds(row0 + bq * K, K)])
    pltpu.sync_copy(e_v.at[pl.ds(0, rem)],
                    acc_sh.at[pl.ds(row0 + nfull * K, rem)])
    plsc.subcore_barrier()

    def _issue_pidx(j, b):
        pltpu.async_copy(pidx_hbm.at[pl.ds(base + j * K, K)],
                         pidx_v[b], sem_p[b])

    def _wait_pidx(b):
        pltpu.make_async_copy(pidx_hbm.at[pl.ds(base, K)],
                              pidx_v[b], sem_p[b]).wait()

    def _unpack_src(b):
        for i in range(K // 16):
            w = pidx_v[b][pl.ds(i * 16, 16)]
            src_v[b][pl.ds(i * 16, 16)] = w & 0xFFFF

    _issue_pidx(0, 0)
    _issue_pidx(1, 1)
    _wait_pidx(0)
    _unpack_src(0)
    pltpu.async_copy(tnode_hbm.at[src_v0], t_v0, sem_t0)
    pltpu.async_copy(efeat_hbm.at[pl.ds(base, K)], e_v, sem_e)

    @pl.loop(0, NCH, step=2)
    def _edge_block(j):
        for parity in range(2):
            jj = j + parity
            b, nb = parity, 1 - parity

            @pl.when(jj + 1 < NCH)
            def _prefetch():
                _wait_pidx(nb)
                _unpack_src(nb)
                pltpu.async_copy(tnode_hbm.at[src_v[nb]], t_v[nb], sem_t[nb])

            pltpu.make_async_copy(tnode_hbm.at[src_v[b]],
                                  t_v[b], sem_t[b]).wait()
            pltpu.make_async_copy(efeat_hbm.at[pl.ds(base, K)],
                                  e_v, sem_e).wait()

            @pl.loop(0, K)
            def _relu_rows(r):
                for gg in range(EMB // 16):
                    sl = pl.ds(gg * 16, 16)
                    e_v[r, sl] = jnp.maximum(t_v[b][r, sl] + e_v[r, sl], 0.0)

            for q in range(K // QROWS):
                for i in range(QROWS // 16):
                    w = pidx_v[b][pl.ds(q * QROWS + i * 16, 16)]
                    dst_q[pl.ds(i * 16, 16)] = jax.lax.shift_right_logical(
                        w, 16)
                pltpu.sync_copy(e_v.at[pl.ds(q * QROWS, QROWS)],
                                acc_sh.at[dst_q], add=True)

            @pl.when(jj + 1 < NCH)
            def _refill_e():
                pltpu.async_copy(efeat_hbm.at[pl.ds(base + (jj + 1) * K, K)],
                                 e_v, sem_e)

            @pl.when(jj + 2 < NCH)
            def _refill_pidx():
                _issue_pidx(jj + 2, b)

    plsc.subcore_barrier()
    pltpu.sync_copy(acc_sh.at[pl.ds(row0, ROWS_PER_TILE)],
                    part_hbm.at[c].at[pl.ds(row0, ROWS_PER_TILE)])


_sc_scatter = functools.partial(
    pl.kernel,
    out_type=jax.ShapeDtypeStruct((NC, N_ACC, EMB), jnp.float32),
    mesh=plsc.VectorSubcoreMesh(core_axis_name="c", subcore_axis_name="s",
                                num_cores=NC, num_subcores=NS),
    scratch_types=[
        pltpu.VMEM((K,), jnp.int32),
        pltpu.VMEM((K,), jnp.int32),
        pltpu.VMEM((K,), jnp.int32),
        pltpu.VMEM((K,), jnp.int32),
        pltpu.VMEM((QROWS,), jnp.int32),
        pltpu.VMEM((K, EMB), jnp.float32),
        pltpu.VMEM((K, EMB), jnp.float32),
        pltpu.VMEM((K, EMB), jnp.float32),
        pltpu.VMEM_SHARED((N_ACC, EMB), jnp.float32),
        pltpu.SemaphoreType.DMA,
        pltpu.SemaphoreType.DMA,
        pltpu.SemaphoreType.DMA,
        pltpu.SemaphoreType.DMA,
        pltpu.SemaphoreType.DMA,
    ],
)(_sc_body)


def _final_body(p_ref, bc_ref, w_ref, b_ref, g_ref, beta_ref, o_ref):
    x = p_ref[0, :N_NODES, :] + p_ref[1, :N_NODES, :] + bc_ref[...]
    y = jnp.dot(x, w_ref[...], preferred_element_type=jnp.float32) + b_ref[...]
    m = jnp.mean(y, axis=-1, keepdims=True)
    d = y - m
    var = jnp.mean(d * d, axis=-1, keepdims=True)
    y = d * jax.lax.rsqrt(var + 1e-5) * g_ref[...] + beta_ref[...]
    o_ref[...] = jnp.maximum(y, 0.0)


def kernel(node_feature_view, augmented_view, edge_index, edge_attr,
           edge_time_emb, boundary_condition, msg_W, msg_b, lin_W, lin_b,
           ln_g, ln_beta):
    E = edge_index.shape[1]
    pad = E_PAD - E

    w_node_t = msg_W[:, :EMB].T
    w_attr_t = msg_W[:, EMB:EMB + 16].T
    w_time_t = msg_W[:, EMB + 16:].T
    b_perm = msg_b

    src = edge_index[0].astype(jnp.int32)
    dst = edge_index[1].astype(jnp.int32)
    pidx = jnp.pad(src | (dst << 16), (0, pad))

    t_node = pl.pallas_call(
        _tnode_body,
        out_shape=jax.ShapeDtypeStruct((N_NODES, EMB), jnp.float32),
    )(node_feature_view, w_node_t)

    e_feat = pl.pallas_call(
        _efeat_body,
        grid=(E_PAD // EB,),
        in_specs=[
            pl.BlockSpec((EB, 16), lambda i: (i, 0)),
            pl.BlockSpec((EB, 16), lambda i: (i, 0)),
            pl.BlockSpec((16, EMB), lambda i: (0, 0)),
            pl.BlockSpec((16, EMB), lambda i: (0, 0)),
            pl.BlockSpec((EMB,), lambda i: (0,)),
        ],
        out_specs=pl.BlockSpec((EB, EMB), lambda i: (i, 0)),
        out_shape=jax.ShapeDtypeStruct((E_PAD, EMB), jnp.float32),
    )(edge_attr, edge_time_emb, w_attr_t, w_time_t, b_perm)

    part = _sc_scatter(t_node, e_feat, pidx)

    out = pl.pallas_call(
        _final_body,
        out_shape=jax.ShapeDtypeStruct((N_NODES, EMB), jnp.float32),
    )(part, boundary_condition, lin_W.T, lin_b, ln_g, ln_beta)
    return out


# speedup vs baseline: 1.0665x; 1.0665x over previous
import functools

import jax
import jax.numpy as jnp
from jax import lax
from jax.experimental import pallas as pl
from jax.experimental.pallas import tpu as pltpu
from jax.experimental.pallas import tpu_sc as plsc

EMB = 128
D_EDGE = 32
N_NODES = 10000
N_EDGES = 320000

NC = 2
NS = 16
NW = NC * NS
K = 64
E_PAD = 327680
EPW = E_PAD // NW
NCH = EPW // K
N_ACC = 10240
ROWS_PER_TILE = N_ACC // NS
DUMMY_DST = N_NODES


def _tnode_body(x_ref, w_ref, o_ref):
    o_ref[...] = jnp.dot(x_ref[...], w_ref[...],
                         preferred_element_type=jnp.float32)


def _efeat_body(x_ref, w_ref, b_ref, o_ref):
    o_ref[...] = jnp.dot(x_ref[...], w_ref[...],
                         preferred_element_type=jnp.float32) + b_ref[...]


def _sc_body(tnode_hbm, efeat_hbm, src_hbm, dst_hbm, part_hbm,
             src_all, dst_v0, dst_v1, t_v0, t_v1, e_v0, e_v1, acc_sh,
             sem_t0, sem_t1, sem_e0, sem_e1, sem_d0, sem_d1):
    c = lax.axis_index("c")
    s = lax.axis_index("s")
    wid = c * NS + s
    t_v = (t_v0, t_v1)
    e_v = (e_v0, e_v1)
    dst_v = (dst_v0, dst_v1)
    sem_t = (sem_t0, sem_t1)
    sem_e = (sem_e0, sem_e1)
    sem_d = (sem_d0, sem_d1)

    pltpu.sync_copy(src_hbm.at[wid], src_all)

    zvec = jnp.zeros((16,), jnp.float32)

    @pl.loop(0, K)
    def _zero_rows(r):
        for cc in range(EMB // 16):
            t_v0[r, pl.ds(cc * 16, 16)] = zvec

    row0 = s * ROWS_PER_TILE
    for b in range(ROWS_PER_TILE // K):
        pltpu.sync_copy(t_v0, acc_sh.at[pl.ds(row0 + b * K, K)])
    plsc.subcore_barrier()

    base = wid * EPW

    pltpu.async_copy(tnode_hbm.at[src_all.at[pl.ds(0, K)]], t_v0, sem_t0)
    pltpu.async_copy(efeat_hbm.at[pl.ds(base, K)], e_v0, sem_e0)
    pltpu.async_copy(dst_hbm.at[wid * NCH], dst_v0, sem_d0)

    @pl.loop(0, NCH, step=2)
    def _edge_block(j):
        for parity in range(2):
            jj = j + parity
            b, nb = parity, 1 - parity

            @pl.when(jj + 1 < NCH)
            def _prefetch():
                jn = jj + 1
                pltpu.async_copy(tnode_hbm.at[src_all.at[pl.ds(jn * K, K)]],
                                 t_v[nb], sem_t[nb])
                pltpu.async_copy(efeat_hbm.at[pl.ds(base + jn * K, K)],
                                 e_v[nb], sem_e[nb])
                pltpu.async_copy(dst_hbm.at[wid * NCH + jn],
                                 dst_v[nb], sem_d[nb])

            pltpu.make_async_copy(tnode_hbm.at[src_all.at[pl.ds(0, K)]],
                                  t_v[b], sem_t[b]).wait()
            pltpu.make_async_copy(efeat_hbm.at[pl.ds(base, K)],
                                  e_v[b], sem_e[b]).wait()
            pltpu.make_async_copy(dst_hbm.at[wid * NCH],
                                  dst_v[b], sem_d[b]).wait()

            @pl.loop(0, K)
            def _relu_rows(r):
                for cc in range(EMB // 16):
                    sl = pl.ds(cc * 16, 16)
                    e_v[b][r, sl] = jnp.maximum(t_v[b][r, sl] + e_v[b][r, sl],
                                                0.0)

            pltpu.sync_copy(e_v[b], acc_sh.at[dst_v[b]], add=True)

    plsc.subcore_barrier()
    pltpu.sync_copy(acc_sh.at[pl.ds(row0, ROWS_PER_TILE)],
                    part_hbm.at[c].at[pl.ds(row0, ROWS_PER_TILE)])


_sc_scatter = functools.partial(
    pl.kernel,
    out_type=jax.ShapeDtypeStruct((NC, N_ACC, EMB), jnp.float32),
    mesh=plsc.VectorSubcoreMesh(core_axis_name="c", subcore_axis_name="s",
                                num_cores=NC, num_subcores=NS),
    scratch_types=[
        pltpu.VMEM((EPW,), jnp.int32),
        pltpu.VMEM((K,), jnp.int32),
        pltpu.VMEM((K,), jnp.int32),
        pltpu.VMEM((K, EMB), jnp.float32),
        pltpu.VMEM((K, EMB), jnp.float32),
        pltpu.VMEM((K, EMB), jnp.float32),
        pltpu.VMEM((K, EMB), jnp.float32),
        pltpu.VMEM_SHARED((N_ACC, EMB), jnp.float32),
        pltpu.SemaphoreType.DMA,
        pltpu.SemaphoreType.DMA,
        pltpu.SemaphoreType.DMA,
        pltpu.SemaphoreType.DMA,
        pltpu.SemaphoreType.DMA,
        pltpu.SemaphoreType.DMA,
    ],
)(_sc_body)


def _final_body(p_ref, bc_ref, w_ref, b_ref, g_ref, beta_ref, o_ref):
    x = p_ref[0, :N_NODES, :] + p_ref[1, :N_NODES, :] + bc_ref[...]
    y = jnp.dot(x, w_ref[...], preferred_element_type=jnp.float32) + b_ref[...]
    m = jnp.mean(y, axis=-1, keepdims=True)
    d = y - m
    var = jnp.mean(d * d, axis=-1, keepdims=True)
    y = d * jax.lax.rsqrt(var + 1e-5) * g_ref[...] + beta_ref[...]
    o_ref[...] = jnp.maximum(y, 0.0)


def kernel(node_feature_view, augmented_view, edge_index, edge_attr,
           edge_time_emb, boundary_condition, msg_W, msg_b, lin_W, lin_b,
           ln_g, ln_beta):
    E = edge_index.shape[1]
    pad = E_PAD - E

    w_node_t = msg_W[:, :EMB].T
    w_edge_t = msg_W[:, EMB:].T
    edge_in = jnp.concatenate([edge_attr, edge_time_emb], axis=1)
    edge_in = jnp.pad(edge_in, ((0, pad), (0, 0)))
    src = jnp.pad(edge_index[0].astype(jnp.int32), (0, pad)).reshape(NW, EPW)
    dst = jnp.pad(edge_index[1].astype(jnp.int32), (0, pad),
                  constant_values=DUMMY_DST).reshape(NW * NCH, K)

    t_node = pl.pallas_call(
        _tnode_body,
        out_shape=jax.ShapeDtypeStruct((N_NODES, EMB), jnp.float32),
    )(node_feature_view, w_node_t)

    EB = 8192
    e_feat = pl.pallas_call(
        _efeat_body,
        grid=(E_PAD // EB,),
        in_specs=[
            pl.BlockSpec((EB, D_EDGE), lambda i: (i, 0)),
            pl.BlockSpec((D_EDGE, EMB), lambda i: (0, 0)),
            pl.BlockSpec((EMB,), lambda i: (0,)),
        ],
        out_specs=pl.BlockSpec((EB, EMB), lambda i: (i, 0)),
        out_shape=jax.ShapeDtypeStruct((E_PAD, EMB), jnp.float32),
    )(edge_in, w_edge_t, msg_b)

    part = _sc_scatter(t_node, e_feat, src, dst)

    out = pl.pallas_call(
        _final_body,
        out_shape=jax.ShapeDtypeStruct((N_NODES, EMB), jnp.float32),
    )(part, boundary_condition, lin_W.T, lin_b, ln_g, ln_beta)
    return out
